# baseline (speedup 1.0000x reference)
import jax
import jax.numpy as jnp
from jax.experimental import pallas as pl


_BT = 256


def _add_block(x_ref, pe_ref, o_ref):
    o_ref[...] = x_ref[...] + pe_ref[...]


def kernel(x, pos_embed):
    B, T, D = x.shape
    grid = (T // _BT,)
    return pl.pallas_call(
        _add_block,
        grid=grid,
        in_specs=[
            pl.BlockSpec((B, _BT, D), lambda i: (0, i, 0)),
            pl.BlockSpec((1, _BT, D), lambda i: (0, i, 0)),
        ],
        out_specs=pl.BlockSpec((B, _BT, D), lambda i: (0, i, 0)),
        out_shape=jax.ShapeDtypeStruct((B, T, D), x.dtype),
    )(x, pos_embed[None])


# speedup vs baseline: 1.0202x; 1.0202x over previous
import jax
import jax.numpy as jnp
from jax.experimental import pallas as pl


_BT = 512


def _add_block(x_ref, pe_ref, o_ref):
    o_ref[...] = x_ref[...] + pe_ref[...]


def kernel(x, pos_embed):
    B, T, D = x.shape
    grid = (T // _BT,)
    return pl.pallas_call(
        _add_block,
        grid=grid,
        in_specs=[
            pl.BlockSpec((B, _BT, D), lambda i: (0, i, 0)),
            pl.BlockSpec((1, _BT, D), lambda i: (0, i, 0)),
        ],
        out_specs=pl.BlockSpec((B, _BT, D), lambda i: (0, i, 0)),
        out_shape=jax.ShapeDtypeStruct((B, T, D), x.dtype),
    )(x, pos_embed[None])
